# baseline (speedup 1.0000x reference)
import functools

import jax
import jax.numpy as jnp
from jax import lax
from jax.experimental import pallas as pl
from jax.experimental.pallas import tpu as pltpu
from jax.experimental.pallas import tpu_sc as plsc

NW = 32
NC = 2
LANES = 16
L = 200
D = 16
CHUNK = 8


def _sc_gather_call(B):
    rpw = B // NW
    n_chunks = rpw // CHUNK
    cl = CHUNK * L
    mesh = plsc.VectorSubcoreMesh(core_axis_name="c", subcore_axis_name="s",
                                  num_cores=NC, num_subcores=NW // NC)

    @functools.partial(
        pl.kernel,
        out_type=jax.ShapeDtypeStruct((14 * B, D), jnp.float32),
        mesh=mesh,
        compiler_params=pltpu.CompilerParams(use_tc_tiling_on_sc=False),
        scratch_types=[
            pltpu.VMEM((cl,), jnp.int32),
            pltpu.VMEM((cl,), jnp.int32),
            pltpu.VMEM((cl,), jnp.int32),
            pltpu.VMEM((cl, D), jnp.float32),
            pltpu.VMEM((cl, D), jnp.float32),
            pltpu.VMEM((cl, D), jnp.float32),
            pltpu.VMEM((CHUNK, D), jnp.float32),
            pltpu.VMEM((CHUNK, D), jnp.float32),
            pltpu.VMEM((CHUNK, D), jnp.float32),
            pltpu.VMEM((cl + LANES,), jnp.float32),
            pltpu.VMEM((rpw,), jnp.int32),
            pltpu.VMEM((rpw, D), jnp.float32),
            pltpu.SemaphoreType.DMA,
            pltpu.SemaphoreType.DMA,
            pltpu.SemaphoreType.DMA,
        ],
    )
    def sc_kernel(sq0, sq1, sq2, tu0, tu1, tu2, tu3, tu4, ti0, ti1, ti2,
                  td0, td1, td2,
                  s0, s1, s2, it0, it1, it2, u0, u1, u2, u3, u4,
                  out,
                  idx0, idx1, idx2, r0, r1, r2, p0, p1, p2,
                  maskf, sidx, srows, sem0, sem1, sem2):
        wid = lax.axis_index("s") * NC + lax.axis_index("c")
        rbase = wid * rpw
        lane = lax.iota(jnp.int32, LANES)
        zero16f = jnp.zeros((LANES,), jnp.float32)

        def chunk_body(g, carry):
            base = rbase + g * CHUNK
            pltpu.sync_copy(sq0.at[pl.ds(base * L, cl)], idx0)
            pltpu.sync_copy(sq1.at[pl.ds(base * L, cl)], idx1)
            pltpu.sync_copy(sq2.at[pl.ds(base * L, cl)], idx2)

            def fix_body(v, c):
                pos = v * LANES
                v0 = idx0[pl.ds(pos, LANES)]
                maskf[pl.ds(pos, LANES)] = jnp.where(
                    v0 == 0, jnp.float32(0.0), jnp.float32(1.0))
                return c
            lax.fori_loop(0, cl // LANES, fix_body, 0)

            d0 = pltpu.async_copy(s0.at[idx0], r0, sem0)
            d1 = pltpu.async_copy(s1.at[idx1], r1, sem1)
            d2 = pltpu.async_copy(s2.at[idx2], r2, sem2)
            d0.wait()
            d1.wait()
            d2.wait()

            def row_body(r, c):
                ro = r * L

                def ab(k, accs):
                    a0, a1, a2 = accs
                    bi = ro + k * LANES
                    mv = maskf[pl.ds(bi, LANES)]
                    for j in range(LANES):
                        i = bi + j
                        mf = mv[j]
                        a0 = a0 + r0[i, :] * mf
                        a1 = a1 + r1[i, :] * mf
                        a2 = a2 + r2[i, :] * mf
                    return (a0, a1, a2)
                a0, a1, a2 = lax.fori_loop(0, L // LANES, ab,
                                           (zero16f, zero16f, zero16f))
                bi = ro + (L // LANES) * LANES
                mv = maskf[pl.ds(bi, LANES)]
                for j in range(L % LANES):
                    i = bi + j
                    mf = mv[j]
                    a0 = a0 + r0[i, :] * mf
                    a1 = a1 + r1[i, :] * mf
                    a2 = a2 + r2[i, :] * mf
                inv = jnp.float32(1.0 / L)
                p0[r, :] = a0 * inv
                p1[r, :] = a1 * inv
                p2[r, :] = a2 * inv
                return c
            lax.fori_loop(0, CHUNK, row_body, 0)

            pltpu.sync_copy(p0, out.at[pl.ds(0 * B + base, CHUNK)])
            pltpu.sync_copy(p1, out.at[pl.ds(1 * B + base, CHUNK)])
            pltpu.sync_copy(p2, out.at[pl.ds(2 * B + base, CHUNK)])
            return carry
        lax.fori_loop(0, n_chunks, chunk_body, 0)

        small = ((ti0, s0, 3), (ti1, s1, 4), (ti2, s2, 5),
                 (td0, it0, 6), (td1, it1, 7), (td2, it2, 8),
                 (tu0, u0, 9), (tu1, u1, 10), (tu2, u2, 11),
                 (tu3, u3, 12), (tu4, u4, 13))
        for idx_hbm, tbl, sec in small:
            pltpu.sync_copy(idx_hbm.at[pl.ds(rbase, rpw)], sidx)
            pltpu.async_copy(tbl.at[sidx], srows, sem0).wait()
            pltpu.sync_copy(srows, out.at[pl.ds(sec * B + rbase, rpw)])

    return sc_kernel


def _ffn_body(x_ref, W1, b1, a1, W2, b2, a2, W3, b3, sig_ref, log_ref):
    x = jnp.concatenate([x_ref[i] for i in range(14)], axis=-1)
    h = jnp.dot(x, W1[...], preferred_element_type=jnp.float32) + b1[0][None, :]
    h = jnp.maximum(h, 0.0) + a1[0][None, :] * jnp.minimum(h, 0.0)
    h = jnp.dot(h, W2[...], preferred_element_type=jnp.float32) + b2[0][None, :]
    h = jnp.maximum(h, 0.0) + a2[0][None, :] * jnp.minimum(h, 0.0)
    logits = jnp.dot(h, W3[...], preferred_element_type=jnp.float32) + b3[0][None, :]
    log_ref[...] = logits
    sig_ref[...] = jax.nn.sigmoid(logits)


def kernel(dense_inputs, target_user_side, seq_inputs, seq_inputs_neg,
           target_item_seq, target_item_side, seq_tables, item_tables,
           user_tables, W1, b1, a1, W2, b2, a2, W3, b3):
    del dense_inputs, seq_inputs_neg
    B = seq_inputs.shape[0]
    si = seq_inputs.astype(jnp.int32)
    sq = [si[:, :, i].reshape(-1) for i in range(3)]
    tu = [target_user_side[:, i].astype(jnp.int32) for i in range(5)]
    ti = [target_item_seq[:, i].astype(jnp.int32) for i in range(3)]
    td = [target_item_side[:, i].astype(jnp.int32) for i in range(3)]

    sc = _sc_gather_call(B)
    feats = sc(sq[0], sq[1], sq[2], tu[0], tu[1], tu[2], tu[3], tu[4],
               ti[0], ti[1], ti[2], td[0], td[1], td[2],
               seq_tables[0], seq_tables[1], seq_tables[2],
               item_tables[0], item_tables[1], item_tables[2],
               user_tables[0], user_tables[1], user_tables[2],
               user_tables[3], user_tables[4])
    feats = feats.reshape(14, B, D)

    sig, logits = pl.pallas_call(
        _ffn_body,
        out_shape=(jax.ShapeDtypeStruct((B, 2), jnp.float32),
                   jax.ShapeDtypeStruct((B, 2), jnp.float32)),
    )(feats, W1, b1.reshape(1, -1), a1.reshape(1, -1),
      W2, b2.reshape(1, -1), a2.reshape(1, -1),
      W3, b3.reshape(1, -1))
    return (sig, logits)
